# Initial kernel scaffold; baseline (speedup 1.0000x reference)
import jax
import jax.numpy as jnp
from jax import lax
from jax.experimental import pallas as pl
from jax.experimental.pallas import tpu as pltpu
from jax.experimental.pallas import tpu_sc as plsc

B = 16384
K = 26
N_OPS = 38461
D = 32
N = B * K
NC, NS = 2, 16
NW = NC * NS
RPW = N // NW
STEP = 128
S = RPW // STEP
GROUP = 8
NG = S // GROUP
GROUP_ROWS = GROUP * STEP


def _sc_gather_body(idx_hbm, table_hbm, out_hbm, idx_v, rows_v, sem):
    c = lax.axis_index("c")
    s = lax.axis_index("s")
    wid = s * NC + c
    base = wid * RPW
    pltpu.sync_copy(idx_hbm.at[wid], idx_v)

    def group(g, carry):
        descs = []
        for i in range(GROUP):
            descs.append(pltpu.async_copy(
                table_hbm.at[idx_v.at[g * GROUP + i]],
                rows_v.at[pl.ds(i * STEP, STEP)],
                sem))
        for d in descs:
            d.wait()
        pltpu.sync_copy(rows_v, out_hbm.at[pl.ds(base + g * GROUP_ROWS, GROUP_ROWS)])
        return carry

    lax.fori_loop(0, NG, group, 0)


def _sc_gather(idx3, table):
    mesh = plsc.VectorSubcoreMesh(
        core_axis_name="c", subcore_axis_name="s", num_cores=NC, num_subcores=NS)
    return pl.kernel(
        _sc_gather_body,
        out_type=jax.ShapeDtypeStruct((N, D), jnp.float32),
        mesh=mesh,
        scratch_types=[
            pltpu.VMEM((S, STEP), jnp.int32),
            pltpu.VMEM((GROUP_ROWS, D), jnp.float32),
            pltpu.SemaphoreType.DMA,
        ],
    )(idx3, table)


def _mlp_body(x_ref, w1_ref, b1_ref, w2_ref, b2_ref, o_ref):
    x = x_ref[...]
    h = jnp.dot(x, w1_ref[...], preferred_element_type=jnp.float32) + b1_ref[...]
    h = 0.5 * h * (1.0 + lax.erf(h * 0.7071067811865476))
    o_ref[...] = jnp.dot(h, w2_ref[...], preferred_element_type=jnp.float32) + b2_ref[...]


MLP_T = 4096


def _tc_mlp(x, w1, b1, w2, b2):
    grid = (N // MLP_T,)
    return pl.pallas_call(
        _mlp_body,
        grid=grid,
        in_specs=[
            pl.BlockSpec((MLP_T, D), lambda i: (i, 0)),
            pl.BlockSpec((D, D), lambda i: (0, 0)),
            pl.BlockSpec((1, D), lambda i: (0, 0)),
            pl.BlockSpec((D, D), lambda i: (0, 0)),
            pl.BlockSpec((1, D), lambda i: (0, 0)),
        ],
        out_specs=pl.BlockSpec((MLP_T, D), lambda i: (i, 0)),
        out_shape=jax.ShapeDtypeStruct((N, D), jnp.float32),
    )(x, w1, b1, w2, b2)


def kernel(attrs, attr_shift, unknown_index, table, W1, b1, W2, b2):
    attrs = attrs.astype(jnp.int32)
    unk = jnp.broadcast_to(unknown_index.astype(jnp.int32)[None, :], attrs.shape)
    idx = jnp.where(attrs < 0, unk, attrs) + attr_shift.astype(jnp.int32)
    idx3 = idx.reshape(NW, S, STEP)
    gathered = _sc_gather(idx3, table)
    out = _tc_mlp(gathered, W1, b1.reshape(1, D), W2, b2.reshape(1, D))
    return out.reshape(B, K, D)


# speedup vs baseline: 9.5320x; 9.5320x over previous
import jax
import jax.numpy as jnp
from jax import lax
from jax.experimental import pallas as pl
from jax.experimental.pallas import tpu as pltpu
from jax.experimental.pallas import tpu_sc as plsc

B = 16384
K = 26
N_OPS = 38461
D = 32
N = B * K
NC, NS = 2, 16
NW = NC * NS
RPW = N // NW
STEP = 128
S = RPW // STEP
GROUP = 8
NG = S // GROUP
GROUP_ROWS = GROUP * STEP


def _sc_gather_body(idx_hbm, table_hbm, out_hbm, idx_v, rows_v, sem):
    c = lax.axis_index("c")
    s = lax.axis_index("s")
    wid = s * NC + c
    base = wid * RPW
    pltpu.sync_copy(idx_hbm.at[wid], idx_v)

    def group(g, carry):
        descs = []
        for i in range(GROUP):
            descs.append(pltpu.async_copy(
                table_hbm.at[idx_v.at[g * GROUP + i]],
                rows_v.at[pl.ds(i * STEP, STEP)],
                sem))
        for d in descs:
            d.wait()
        pltpu.sync_copy(rows_v, out_hbm.at[pl.ds(base + g * GROUP_ROWS, GROUP_ROWS)])
        return carry

    lax.fori_loop(0, NG, group, 0)


def _sc_gather(idx3, table):
    mesh = plsc.VectorSubcoreMesh(
        core_axis_name="c", subcore_axis_name="s", num_cores=NC, num_subcores=NS)
    return pl.kernel(
        _sc_gather_body,
        out_type=jax.ShapeDtypeStruct((N, D), jnp.float32),
        mesh=mesh,
        scratch_types=[
            pltpu.VMEM((S, STEP), jnp.int32),
            pltpu.VMEM((GROUP_ROWS, D), jnp.float32),
            pltpu.SemaphoreType.DMA,
        ],
        compiler_params=pltpu.CompilerParams(use_tc_tiling_on_sc=False),
    )(idx3, table)


def _mlp_body(x_ref, w1_ref, b1_ref, w2_ref, b2_ref, o_ref):
    x = x_ref[...]
    h = jnp.dot(x, w1_ref[...], preferred_element_type=jnp.float32) + b1_ref[...]
    h = 0.5 * h * (1.0 + lax.erf(h * 0.7071067811865476))
    o_ref[...] = jnp.dot(h, w2_ref[...], preferred_element_type=jnp.float32) + b2_ref[...]


MLP_T = 4096


def _tc_mlp(x, w1, b1, w2, b2):
    grid = (N // MLP_T,)
    return pl.pallas_call(
        _mlp_body,
        grid=grid,
        in_specs=[
            pl.BlockSpec((MLP_T, D), lambda i: (i, 0)),
            pl.BlockSpec((D, D), lambda i: (0, 0)),
            pl.BlockSpec((1, D), lambda i: (0, 0)),
            pl.BlockSpec((D, D), lambda i: (0, 0)),
            pl.BlockSpec((1, D), lambda i: (0, 0)),
        ],
        out_specs=pl.BlockSpec((MLP_T, D), lambda i: (i, 0)),
        out_shape=jax.ShapeDtypeStruct((N, D), jnp.float32),
    )(x, w1, b1, w2, b2)


def kernel(attrs, attr_shift, unknown_index, table, W1, b1, W2, b2):
    attrs = attrs.astype(jnp.int32)
    unk = jnp.broadcast_to(unknown_index.astype(jnp.int32)[None, :], attrs.shape)
    idx = jnp.where(attrs < 0, unk, attrs) + attr_shift.astype(jnp.int32)
    idx3 = idx.reshape(NW, S, STEP)
    gathered = _sc_gather(idx3, table)
    out = _tc_mlp(gathered, W1, b1.reshape(1, D), W2, b2.reshape(1, D))
    return out.reshape(B, K, D)


# speedup vs baseline: 10.8527x; 1.1386x over previous
import jax
import jax.numpy as jnp
from jax import lax
from jax.experimental import pallas as pl
from jax.experimental.pallas import tpu as pltpu
from jax.experimental.pallas import tpu_sc as plsc

B = 16384
K = 26
N_OPS = 38461
D = 32
N = B * K
NC, NS = 2, 16
NW = NC * NS
RPW = N // NW
STEP = 128
S = RPW // STEP
GROUP = 8
NG = S // GROUP
GR = GROUP * STEP


def _mlp_body(x_ref, w1t_ref, b1_ref, w2t_ref, b2_ref, o_ref):
    x = x_ref[...]
    h = jnp.dot(w1t_ref[...], x, preferred_element_type=jnp.float32) + b1_ref[...]
    h = 0.5 * h * (1.0 + lax.erf(h * 0.7071067811865476))
    y = jnp.dot(w2t_ref[...], h, preferred_element_type=jnp.float32) + b2_ref[...]
    o_ref[...] = y.T


MLP_TB = 8192


def _tc_mlp_table(table_t, w1t, b1c, w2t, b2c):
    total = table_t.shape[1]
    grid = (pl.cdiv(total, MLP_TB),)
    return pl.pallas_call(
        _mlp_body,
        grid=grid,
        in_specs=[
            pl.BlockSpec((D, MLP_TB), lambda i: (0, i)),
            pl.BlockSpec((D, D), lambda i: (0, 0)),
            pl.BlockSpec((D, 1), lambda i: (0, 0)),
            pl.BlockSpec((D, D), lambda i: (0, 0)),
            pl.BlockSpec((D, 1), lambda i: (0, 0)),
        ],
        out_specs=pl.BlockSpec((MLP_TB, D), lambda i: (i, 0)),
        out_shape=jax.ShapeDtypeStruct((total, D), jnp.float32),
    )(table_t, w1t, b1c, w2t, b2c)


def _sc_gather_body(idx_hbm, table_hbm, out_hbm, idx_v, rows_v, rows_t, sem):
    c = lax.axis_index("c")
    s = lax.axis_index("s")
    wid = s * NC + c
    base = wid * RPW
    pltpu.sync_copy(idx_hbm.at[wid], idx_v)

    lane = lax.iota(jnp.int32, 16)

    def group(g, carry):
        descs = []
        for i in range(GROUP):
            descs.append(pltpu.async_copy(
                table_hbm.at[idx_v.at[g * GROUP + i]],
                rows_v.at[pl.ds(i * STEP, STEP)],
                sem))
        for d in descs:
            d.wait()

        def tchunk(tc, carry2):
            t0 = tc * 16
            row_ids = t0 + lane
            for j in range(D):
                vals = plsc.load_gather(rows_v, [row_ids, jnp.full((16,), j, jnp.int32)])
                rows_t[j, pl.ds(t0, 16)] = vals
            return carry2

        lax.fori_loop(0, GR // 16, tchunk, 0)

        n0 = base + g * GR
        k = n0 // B
        t0 = n0 % B
        pltpu.sync_copy(rows_t, out_hbm.at[k, :, pl.ds(t0, GR)])
        return carry

    lax.fori_loop(0, NG, group, 0)


def _sc_gather(idx3, table_flat):
    mesh = plsc.VectorSubcoreMesh(
        core_axis_name="c", subcore_axis_name="s", num_cores=NC, num_subcores=NS)
    return pl.kernel(
        _sc_gather_body,
        out_type=jax.ShapeDtypeStruct((K, D, B), jnp.float32),
        mesh=mesh,
        scratch_types=[
            pltpu.VMEM((S, STEP), jnp.int32),
            pltpu.VMEM((GR, D), jnp.float32),
            pltpu.VMEM((D, GR), jnp.float32),
            pltpu.SemaphoreType.DMA,
        ],
        compiler_params=pltpu.CompilerParams(
            use_tc_tiling_on_sc=False, needs_layout_passes=False),
    )(idx3, table_flat)


def kernel(attrs, attr_shift, unknown_index, table, W1, b1, W2, b2):
    attrs_t = attrs.T.astype(jnp.int32)
    unk = unknown_index.astype(jnp.int32)[:, None]
    idx_t = jnp.where(attrs_t < 0, unk, attrs_t) + attr_shift.astype(jnp.int32).T
    idx3 = idx_t.reshape(NW, S, STEP)

    table2 = _tc_mlp_table(table.T, W1.T, b1.reshape(D, 1), W2.T, b2.reshape(D, 1))

    out3 = _sc_gather(idx3, table2)
    return out3.transpose(2, 0, 1)


# speedup vs baseline: 21.8201x; 2.0106x over previous
import jax
import jax.numpy as jnp
from jax import lax
from jax.experimental import pallas as pl
from jax.experimental.pallas import tpu as pltpu
from jax.experimental.pallas import tpu_sc as plsc

B = 16384
K = 26
N_OPS = 38461
D = 32
N = B * K
TOTAL = K * N_OPS
NC, NS = 2, 16
NW = NC * NS
RPW = N // NW
STEP = 128
S = RPW // STEP
GROUP = 8
NG = S // GROUP
GR = GROUP * STEP

MLP_TB = 8192
QB = MLP_TB // 4
NBLK = pl.cdiv(TOTAL, MLP_TB)
T4 = NBLK * MLP_TB


def _mlp_body(x_ref, w1_ref, b1_ref, w2_ref, b2_ref, o_ref):
    x = x_ref[...]
    x128 = jnp.concatenate([x[:, c * QB:(c + 1) * QB] for c in range(4)], axis=0)
    h = jnp.dot(w1_ref[...], x128, preferred_element_type=jnp.float32) + b1_ref[...]
    h = 0.5 * h * (1.0 + lax.erf(h * 0.7071067811865476))
    y = jnp.dot(w2_ref[...], h, preferred_element_type=jnp.float32) + b2_ref[...]
    o_ref[...] = y.T


def _tc_mlp_table(table_t, w1bd, b1bd, w2bd, b2bd):
    return pl.pallas_call(
        _mlp_body,
        grid=(NBLK,),
        in_specs=[
            pl.BlockSpec((D, MLP_TB), lambda i: (0, i)),
            pl.BlockSpec((128, 128), lambda i: (0, 0)),
            pl.BlockSpec((128, 1), lambda i: (0, 0)),
            pl.BlockSpec((128, 128), lambda i: (0, 0)),
            pl.BlockSpec((128, 1), lambda i: (0, 0)),
        ],
        out_specs=pl.BlockSpec((QB, 128), lambda i: (i, 0)),
        out_shape=jax.ShapeDtypeStruct((NBLK * QB, 128), jnp.float32),
    )(table_t, w1bd, b1bd, w2bd, b2bd)


def _sc_gather_body(idx_hbm, table_hbm, out_hbm,
                    idx_v, buf_a, buf_b, rows_t, sga, sgb):
    c = lax.axis_index("c")
    s = lax.axis_index("s")
    wid = s * NC + c
    base = wid * RPW
    pltpu.sync_copy(idx_hbm.at[wid], idx_v)

    lane = lax.iota(jnp.int32, 16)
    lane16 = lane + 16

    def fire(g, buf, sem):
        for i in range(GROUP):
            pltpu.async_copy(
                table_hbm.at[idx_v.at[g * GROUP + i]],
                buf.at[pl.ds(i * STEP, STEP)],
                sem)

    def drain(g, buf, sem):
        for i in range(GROUP):
            pltpu.make_async_copy(
                table_hbm.at[idx_v.at[g * GROUP + i]],
                buf.at[pl.ds(i * STEP, STEP)],
                sem).wait()

    def transpose_wb(g, buf):
        def tr(tt, carry):
            for u in range(8):
                t = tt * 8 + u
                v0 = buf[t, pl.ds(0, 16)]
                v1 = buf[t, pl.ds(16, 16)]
                tcol = jnp.zeros((16,), jnp.int32) + t
                plsc.store_scatter(rows_t, [lane, tcol], v0)
                plsc.store_scatter(rows_t, [lane16, tcol], v1)
            return carry

        lax.fori_loop(0, GR // 8, tr, 0)
        n0 = base + g * GR
        k = n0 // B
        t0 = n0 % B
        pltpu.sync_copy(rows_t, out_hbm.at[k, :, pl.ds(t0, GR)])

    fire(0, buf_a, sga)

    def pair(u, carry):
        ga = 2 * u
        fire(ga + 1, buf_b, sgb)
        drain(ga, buf_a, sga)
        transpose_wb(ga, buf_a)
        fire(ga + 2, buf_a, sga)
        drain(ga + 1, buf_b, sgb)
        transpose_wb(ga + 1, buf_b)
        return carry

    lax.fori_loop(0, (NG - 1) // 2, pair, 0)
    drain(NG - 1, buf_a, sga)
    transpose_wb(NG - 1, buf_a)


def _sc_gather(idx3, table4):
    mesh = plsc.VectorSubcoreMesh(
        core_axis_name="c", subcore_axis_name="s", num_cores=NC, num_subcores=NS)
    return pl.kernel(
        _sc_gather_body,
        out_type=jax.ShapeDtypeStruct((K, D, B), jnp.float32),
        mesh=mesh,
        scratch_types=[
            pltpu.VMEM((S, STEP), jnp.int32),
            pltpu.VMEM((GR, D), jnp.float32),
            pltpu.VMEM((GR, D), jnp.float32),
            pltpu.VMEM((D, GR), jnp.float32),
            pltpu.SemaphoreType.DMA,
            pltpu.SemaphoreType.DMA,
        ],
        compiler_params=pltpu.CompilerParams(
            use_tc_tiling_on_sc=False, needs_layout_passes=False),
    )(idx3, table4)


def kernel(attrs, attr_shift, unknown_index, table, W1, b1, W2, b2):
    attrs_t = attrs.T.astype(jnp.int32)
    unk = unknown_index.astype(jnp.int32)[:, None]
    r = jnp.where(attrs_t < 0, unk, attrs_t) + attr_shift.astype(jnp.int32).T
    rho = ((r >> 13) << 13) + ((r & 2047) << 2) + ((r >> 11) & 3)
    idx3 = rho.reshape(NW, S, STEP)

    eye4 = jnp.eye(4, dtype=jnp.float32)
    w1bd = jnp.kron(eye4, W1.T)
    w2bd = jnp.kron(eye4, W2.T)
    b1bd = jnp.tile(b1, 4).reshape(128, 1)
    b2bd = jnp.tile(b2, 4).reshape(128, 1)

    table2 = _tc_mlp_table(table.T, w1bd, b1bd, w2bd, b2bd)
    table4 = table2.reshape(T4, D)

    out3 = _sc_gather(idx3, table4)
    return out3.transpose(2, 0, 1)


# speedup vs baseline: 24.4773x; 1.1218x over previous
import jax
import jax.numpy as jnp
from jax import lax
from jax.experimental import pallas as pl
from jax.experimental.pallas import tpu as pltpu
from jax.experimental.pallas import tpu_sc as plsc

B = 16384
K = 26
N_OPS = 38461
D = 32
N = B * K
TOTAL = K * N_OPS
NC, NS = 2, 16
NW = NC * NS
RPW = N // NW
STEP = 128
S = RPW // STEP
GROUP = 8
NG = S // GROUP
GR = GROUP * STEP

MLP_TB = 8192
QB = MLP_TB // 4
NBLK = pl.cdiv(TOTAL, MLP_TB)
T4 = NBLK * MLP_TB


def _mlp_body(x_ref, w1_ref, b1_ref, w2_ref, b2_ref, o_ref):
    x = x_ref[...]
    x128 = jnp.concatenate([x[:, c * QB:(c + 1) * QB] for c in range(4)], axis=0)
    h = jnp.dot(w1_ref[...], x128, preferred_element_type=jnp.float32) + b1_ref[...]
    h = 0.5 * h * (1.0 + lax.erf(h * 0.7071067811865476))
    y = jnp.dot(w2_ref[...], h, preferred_element_type=jnp.float32) + b2_ref[...]
    o_ref[...] = y.T


def _tc_mlp_table(table_t, w1bd, b1bd, w2bd, b2bd):
    return pl.pallas_call(
        _mlp_body,
        grid=(NBLK,),
        in_specs=[
            pl.BlockSpec((D, MLP_TB), lambda i: (0, i)),
            pl.BlockSpec((128, 128), lambda i: (0, 0)),
            pl.BlockSpec((128, 1), lambda i: (0, 0)),
            pl.BlockSpec((128, 128), lambda i: (0, 0)),
            pl.BlockSpec((128, 1), lambda i: (0, 0)),
        ],
        out_specs=pl.BlockSpec((QB, 128), lambda i: (i, 0)),
        out_shape=jax.ShapeDtypeStruct((NBLK * QB, 128), jnp.float32),
    )(table_t, w1bd, b1bd, w2bd, b2bd)


def _sc_gather_body(idx_hbm, table_hbm, out_hbm,
                    idx_v, buf_a, buf_b, rows_t, sga, sgb):
    c = lax.axis_index("c")
    s = lax.axis_index("s")
    wid = s * NC + c
    base = wid * RPW
    pltpu.sync_copy(idx_hbm.at[wid], idx_v)

    lane = lax.iota(jnp.int32, 16)
    lane16 = lane + 16
    fpart0 = (lane >> 3) * 8192 + (lane & 7) * 128
    fpart1 = (lane16 >> 3) * 8192 + (lane16 & 7) * 128

    def fire(g, buf, sem):
        for i in range(GROUP):
            pltpu.async_copy(
                table_hbm.at[idx_v.at[g * GROUP + i]],
                buf.at[pl.ds(i * STEP, STEP)],
                sem)

    def drain(g, buf, sem):
        for i in range(GROUP):
            pltpu.make_async_copy(
                table_hbm.at[idx_v.at[g * GROUP + i]],
                buf.at[pl.ds(i * STEP, STEP)],
                sem).wait()

    def transpose_wb(g, buf):
        def tr(tt, carry):
            for u in range(8):
                t = tt * 8 + u
                v0 = buf[t, pl.ds(0, 16)]
                v1 = buf[t, pl.ds(16, 16)]
                tpart = ((t >> 7) << 10) + (t & 127)
                plsc.store_scatter(rows_t, [fpart0 + tpart], v0)
                plsc.store_scatter(rows_t, [fpart1 + tpart], v1)
            return carry

        lax.fori_loop(0, GR // 8, tr, 0)
        n0 = base + g * GR
        k = n0 // B
        w0 = (n0 % B) * 8
        for jt in range(4):
            pltpu.sync_copy(rows_t.at[pl.ds(jt * 8192, 8192)],
                            out_hbm.at[k, pl.ds(jt * 131072 + w0, 8192)])

    fire(0, buf_a, sga)

    def pair(u, carry):
        ga = 2 * u
        fire(ga + 1, buf_b, sgb)
        drain(ga, buf_a, sga)
        transpose_wb(ga, buf_a)
        fire(ga + 2, buf_a, sga)
        drain(ga + 1, buf_b, sgb)
        transpose_wb(ga + 1, buf_b)
        return carry

    lax.fori_loop(0, (NG - 1) // 2, pair, 0)
    drain(NG - 1, buf_a, sga)
    transpose_wb(NG - 1, buf_a)


def _sc_gather(idx3, table4):
    mesh = plsc.VectorSubcoreMesh(
        core_axis_name="c", subcore_axis_name="s", num_cores=NC, num_subcores=NS)
    return pl.kernel(
        _sc_gather_body,
        out_type=jax.ShapeDtypeStruct((K, D * B), jnp.float32),
        mesh=mesh,
        scratch_types=[
            pltpu.VMEM((S, STEP), jnp.int32),
            pltpu.VMEM((GR, D), jnp.float32),
            pltpu.VMEM((GR, D), jnp.float32),
            pltpu.VMEM((D * GR,), jnp.float32),
            pltpu.SemaphoreType.DMA,
            pltpu.SemaphoreType.DMA,
        ],
        compiler_params=pltpu.CompilerParams(
            use_tc_tiling_on_sc=False, needs_layout_passes=False),
    )(idx3, table4)


def kernel(attrs, attr_shift, unknown_index, table, W1, b1, W2, b2):
    attrs_t = attrs.T.astype(jnp.int32)
    unk = unknown_index.astype(jnp.int32)[:, None]
    r = jnp.where(attrs_t < 0, unk, attrs_t) + attr_shift.astype(jnp.int32).T
    rho = ((r >> 13) << 13) + ((r & 2047) << 2) + ((r >> 11) & 3)
    idx3 = rho.reshape(NW, S, STEP)

    eye4 = jnp.eye(4, dtype=jnp.float32)
    w1bd = jnp.kron(eye4, W1.T)
    w2bd = jnp.kron(eye4, W2.T)
    b1bd = jnp.tile(b1, 4).reshape(128, 1)
    b2bd = jnp.tile(b2, 4).reshape(128, 1)

    table2 = _tc_mlp_table(table.T, w1bd, b1bd, w2bd, b2bd)
    table4 = table2.reshape(T4, D)

    out2 = _sc_gather(idx3, table4)
    out5 = out2.reshape(K, 4, B // 128, 8, 128)
    return out5.transpose(2, 4, 0, 1, 3).reshape(B, K, D)


# speedup vs baseline: 28.2513x; 1.1542x over previous
import jax
import jax.numpy as jnp
from jax import lax
from jax.experimental import pallas as pl
from jax.experimental.pallas import tpu as pltpu
from jax.experimental.pallas import tpu_sc as plsc

B = 16384
K = 26
N_OPS = 38461
D = 32
N = B * K
TOTAL = K * N_OPS
NC, NS = 2, 16
NW = NC * NS
RPW = N // NW
STEP = 128
S = RPW // STEP
GROUP = 8
NG = S // GROUP
GR = GROUP * STEP

MLP_TB = 16384
QB = MLP_TB // 8
NBLK = pl.cdiv(TOTAL, MLP_TB)
T4 = NBLK * MLP_TB


def _mlp_body(x_ref, w1_ref, b1_ref, w2_ref, b2_ref, o_ref):
    x = x_ref[...]
    x256 = jnp.concatenate([x[:, c * QB:(c + 1) * QB] for c in range(8)], axis=0)
    h = jnp.dot(w1_ref[...], x256, preferred_element_type=jnp.float32) + b1_ref[...]
    h = 0.5 * h * (1.0 + lax.erf(h * 0.7071067811865476))
    y = jnp.dot(w2_ref[...], h, preferred_element_type=jnp.float32) + b2_ref[...]
    o_ref[...] = jnp.concatenate([y[:128, :].T, y[128:, :].T], axis=0)


def _tc_mlp_table(table_t, w1bd, b1bd, w2bd, b2bd):
    return pl.pallas_call(
        _mlp_body,
        grid=(NBLK,),
        in_specs=[
            pl.BlockSpec((D, MLP_TB), lambda i: (0, i)),
            pl.BlockSpec((256, 256), lambda i: (0, 0)),
            pl.BlockSpec((256, 1), lambda i: (0, 0)),
            pl.BlockSpec((256, 256), lambda i: (0, 0)),
            pl.BlockSpec((256, 1), lambda i: (0, 0)),
        ],
        out_specs=pl.BlockSpec((2 * QB, 128), lambda i: (i, 0)),
        out_shape=jax.ShapeDtypeStruct((NBLK * 2 * QB, 128), jnp.float32),
    )(table_t, w1bd, b1bd, w2bd, b2bd)


def _sc_gather_body(idx_hbm, table_hbm, out_hbm,
                    idx_v, buf_a, buf_b, rows_t, sga, sgb):
    c = lax.axis_index("c")
    s = lax.axis_index("s")
    wid = s * NC + c
    base = wid * RPW
    pltpu.sync_copy(idx_hbm.at[wid], idx_v)

    lane = lax.iota(jnp.int32, 16)
    lane16 = lane + 16
    fpart0 = (lane >> 3) * 8192 + (lane & 7) * 128
    fpart1 = (lane16 >> 3) * 8192 + (lane16 & 7) * 128

    def fire(g, buf, sem):
        for i in range(GROUP):
            pltpu.async_copy(
                table_hbm.at[idx_v.at[g * GROUP + i]],
                buf.at[pl.ds(i * STEP, STEP)],
                sem)

    def drain(g, buf, sem):
        for i in range(GROUP):
            pltpu.make_async_copy(
                table_hbm.at[idx_v.at[g * GROUP + i]],
                buf.at[pl.ds(i * STEP, STEP)],
                sem).wait()

    def transpose_wb(g, buf):
        def tr(tt, carry):
            vals = []
            for u in range(8):
                t = tt * 8 + u
                v0 = buf[t, pl.ds(0, 16)]
                v1 = buf[t, pl.ds(16, 16)]
                tpart = ((t >> 7) << 10) + (t & 127)
                vals.append((fpart0 + tpart, v0, fpart1 + tpart, v1))
            for i0, v0, i1, v1 in vals:
                plsc.store_scatter(rows_t, [i0], v0)
                plsc.store_scatter(rows_t, [i1], v1)
            return carry

        lax.fori_loop(0, GR // 8, tr, 0)
        n0 = base + g * GR
        k = n0 // B
        w0 = (n0 % B) * 8
        for jt in range(4):
            pltpu.sync_copy(rows_t.at[pl.ds(jt * 8192, 8192)],
                            out_hbm.at[k, pl.ds(jt * 131072 + w0, 8192)])

    fire(0, buf_a, sga)

    def pair(u, carry):
        ga = 2 * u
        fire(ga + 1, buf_b, sgb)
        drain(ga, buf_a, sga)
        transpose_wb(ga, buf_a)
        fire(ga + 2, buf_a, sga)
        drain(ga + 1, buf_b, sgb)
        transpose_wb(ga + 1, buf_b)
        return carry

    lax.fori_loop(0, (NG - 1) // 2, pair, 0)
    drain(NG - 1, buf_a, sga)
    transpose_wb(NG - 1, buf_a)


def _sc_gather(idx3, table4):
    mesh = plsc.VectorSubcoreMesh(
        core_axis_name="c", subcore_axis_name="s", num_cores=NC, num_subcores=NS)
    return pl.kernel(
        _sc_gather_body,
        out_type=jax.ShapeDtypeStruct((K, D * B), jnp.float32),
        mesh=mesh,
        scratch_types=[
            pltpu.VMEM((S, STEP), jnp.int32),
            pltpu.VMEM((GR, D), jnp.float32),
            pltpu.VMEM((GR, D), jnp.float32),
            pltpu.VMEM((D * GR,), jnp.float32),
            pltpu.SemaphoreType.DMA,
            pltpu.SemaphoreType.DMA,
        ],
        compiler_params=pltpu.CompilerParams(
            use_tc_tiling_on_sc=False, needs_layout_passes=False),
    )(idx3, table4)


def kernel(attrs, attr_shift, unknown_index, table, W1, b1, W2, b2):
    attrs_t = attrs.T.astype(jnp.int32)
    unk = unknown_index.astype(jnp.int32)[:, None]
    r = jnp.where(attrs_t < 0, unk, attrs_t) + attr_shift.astype(jnp.int32).T
    rho = ((r >> 13) << 13) + ((r & 2047) << 2) + ((r >> 11) & 3)
    idx3 = rho.reshape(NW, S, STEP)

    eye8 = jnp.eye(8, dtype=jnp.float32)
    w1bd = jnp.kron(eye8, W1.T)
    w2bd = jnp.kron(eye8, W2.T)
    b1bd = jnp.tile(b1, 8).reshape(256, 1)
    b2bd = jnp.tile(b2, 8).reshape(256, 1)

    table2 = _tc_mlp_table(table.T, w1bd, b1bd, w2bd, b2bd)
    table4 = table2.reshape(T4, D)

    out2 = _sc_gather(idx3, table4)
    out5 = out2.reshape(K, 4, B // 128, 8, 128)
    return out5.transpose(2, 4, 0, 1, 3).reshape(B, K, D)


# speedup vs baseline: 28.2862x; 1.0012x over previous
import jax
import jax.numpy as jnp
from jax import lax
from jax.experimental import pallas as pl
from jax.experimental.pallas import tpu as pltpu
from jax.experimental.pallas import tpu_sc as plsc

B = 16384
K = 26
N_OPS = 38461
D = 32
N = B * K
TOTAL = K * N_OPS
NC, NS = 2, 16
NW = NC * NS
RPW = N // NW
STEP = 128
S = RPW // STEP
GROUP = 8
NG = S // GROUP
GR = GROUP * STEP

MLP_TB = 16384
QB = MLP_TB // 8
NBLK = pl.cdiv(TOTAL, MLP_TB)
T4 = NBLK * MLP_TB


def _mlp_body(x_ref, w1_ref, b1_ref, w2_ref, b2_ref, o_ref):
    x = x_ref[...]
    x256 = jnp.concatenate([x[:, c * QB:(c + 1) * QB] for c in range(8)], axis=0)
    h = jnp.dot(w1_ref[...], x256, preferred_element_type=jnp.float32) + b1_ref[...]
    h = 0.5 * h * (1.0 + lax.erf(h * 0.7071067811865476))
    y = jnp.dot(w2_ref[...], h, preferred_element_type=jnp.float32) + b2_ref[...]
    o_ref[...] = jnp.concatenate([y[:128, :].T, y[128:, :].T], axis=0)


def _tc_mlp_table(table_t, w1bd, b1bd, w2bd, b2bd):
    return pl.pallas_call(
        _mlp_body,
        grid=(NBLK,),
        in_specs=[
            pl.BlockSpec((D, MLP_TB), lambda i: (0, i)),
            pl.BlockSpec((256, 256), lambda i: (0, 0)),
            pl.BlockSpec((256, 1), lambda i: (0, 0)),
            pl.BlockSpec((256, 256), lambda i: (0, 0)),
            pl.BlockSpec((256, 1), lambda i: (0, 0)),
        ],
        out_specs=pl.BlockSpec((2 * QB, 128), lambda i: (i, 0)),
        out_shape=jax.ShapeDtypeStruct((NBLK * 2 * QB, 128), jnp.float32),
    )(table_t, w1bd, b1bd, w2bd, b2bd)


def _sc_gather_body(idx_hbm, table_hbm, out_hbm,
                    idx_v, buf_a, buf_b, rows_t, sga, sgb):
    c = lax.axis_index("c")
    s = lax.axis_index("s")
    wid = s * NC + c
    base = wid * RPW
    pltpu.sync_copy(idx_hbm.at[wid], idx_v)

    lane = lax.iota(jnp.int32, 16)
    lane16 = lane + 16
    fpart0 = (lane >> 3) * 8192 + (lane & 7) * 128
    fpart1 = (lane16 >> 3) * 8192 + (lane16 & 7) * 128

    def fire(g, buf, sem):
        for i in range(GROUP):
            pltpu.async_copy(
                table_hbm.at[idx_v.at[g * GROUP + i]],
                buf.at[pl.ds(i * STEP, STEP)],
                sem)

    def drain(g, buf, sem):
        for i in range(GROUP):
            pltpu.make_async_copy(
                table_hbm.at[idx_v.at[g * GROUP + i]],
                buf.at[pl.ds(i * STEP, STEP)],
                sem).wait()

    def transpose_wb(g, buf):
        def tr(tt, carry):
            vals = []
            for u in range(16):
                t = tt * 16 + u
                v0 = buf[t, pl.ds(0, 16)]
                v1 = buf[t, pl.ds(16, 16)]
                tpart = ((t >> 7) << 10) + (t & 127)
                vals.append((fpart0 + tpart, v0, fpart1 + tpart, v1))
            for i0, v0, i1, v1 in vals:
                plsc.store_scatter(rows_t, [i0], v0)
                plsc.store_scatter(rows_t, [i1], v1)
            return carry

        lax.fori_loop(0, GR // 16, tr, 0)
        n0 = base + g * GR
        k = n0 // B
        w0 = (n0 % B) * 8
        for jt in range(4):
            pltpu.sync_copy(rows_t.at[pl.ds(jt * 8192, 8192)],
                            out_hbm.at[k, pl.ds(jt * 131072 + w0, 8192)])

    fire(0, buf_a, sga)

    def pair(u, carry):
        ga = 2 * u
        fire(ga + 1, buf_b, sgb)
        drain(ga, buf_a, sga)
        transpose_wb(ga, buf_a)
        fire(ga + 2, buf_a, sga)
        drain(ga + 1, buf_b, sgb)
        transpose_wb(ga + 1, buf_b)
        return carry

    lax.fori_loop(0, (NG - 1) // 2, pair, 0)
    drain(NG - 1, buf_a, sga)
    transpose_wb(NG - 1, buf_a)


def _sc_gather(idx3, table4):
    mesh = plsc.VectorSubcoreMesh(
        core_axis_name="c", subcore_axis_name="s", num_cores=NC, num_subcores=NS)
    return pl.kernel(
        _sc_gather_body,
        out_type=jax.ShapeDtypeStruct((K, D * B), jnp.float32),
        mesh=mesh,
        scratch_types=[
            pltpu.VMEM((S, STEP), jnp.int32),
            pltpu.VMEM((GR, D), jnp.float32),
            pltpu.VMEM((GR, D), jnp.float32),
            pltpu.VMEM((D * GR,), jnp.float32),
            pltpu.SemaphoreType.DMA,
            pltpu.SemaphoreType.DMA,
        ],
        compiler_params=pltpu.CompilerParams(
            use_tc_tiling_on_sc=False, needs_layout_passes=False,
            disable_bounds_checks=True),
    )(idx3, table4)


def kernel(attrs, attr_shift, unknown_index, table, W1, b1, W2, b2):
    attrs_t = attrs.T.astype(jnp.int32)
    unk = unknown_index.astype(jnp.int32)[:, None]
    r = jnp.where(attrs_t < 0, unk, attrs_t) + attr_shift.astype(jnp.int32).T
    rho = ((r >> 13) << 13) + ((r & 2047) << 2) + ((r >> 11) & 3)
    idx3 = rho.reshape(NW, S, STEP)

    eye8 = jnp.eye(8, dtype=jnp.float32)
    w1bd = jnp.kron(eye8, W1.T)
    w2bd = jnp.kron(eye8, W2.T)
    b1bd = jnp.tile(b1, 8).reshape(256, 1)
    b2bd = jnp.tile(b2, 8).reshape(256, 1)

    table2 = _tc_mlp_table(table.T, w1bd, b1bd, w2bd, b2bd)
    table4 = table2.reshape(T4, D)

    out2 = _sc_gather(idx3, table4)
    out5 = out2.reshape(K, 4, B // 128, 8, 128)
    return out5.transpose(2, 4, 0, 1, 3).reshape(B, K, D)


# speedup vs baseline: 40.3735x; 1.4273x over previous
import jax
import jax.numpy as jnp
from jax import lax
from jax.experimental import pallas as pl
from jax.experimental.pallas import tpu as pltpu
from jax.experimental.pallas import tpu_sc as plsc

B = 16384
K = 26
N_OPS = 38461
D = 32
N = B * K
TOTAL = K * N_OPS
NC, NS = 2, 16
NW = NC * NS
RPW = N // NW
STEP = 128
S = RPW // STEP
GROUP = 8
NG = S // GROUP
GR = GROUP * STEP

MLP_TB = 16384
QB = MLP_TB // 8
NBLK = pl.cdiv(TOTAL, MLP_TB)
T4 = NBLK * MLP_TB


def _mlp_body(x_ref, w1_ref, b1_ref, w2_ref, b2_ref, o_ref):
    x = x_ref[...]
    x256 = jnp.concatenate([x[:, c * QB:(c + 1) * QB] for c in range(8)], axis=0)
    h = jnp.dot(w1_ref[...], x256, preferred_element_type=jnp.float32) + b1_ref[...]
    h = 0.5 * h * (1.0 + lax.erf(h * 0.7071067811865476))
    y = jnp.dot(w2_ref[...], h, preferred_element_type=jnp.float32) + b2_ref[...]
    o_ref[...] = jnp.concatenate([y[:128, :].T, y[128:, :].T], axis=0)


def _tc_mlp_table(table_t, w1bd, b1bd, w2bd, b2bd):
    return pl.pallas_call(
        _mlp_body,
        grid=(NBLK,),
        in_specs=[
            pl.BlockSpec((D, MLP_TB), lambda i: (0, i)),
            pl.BlockSpec((256, 256), lambda i: (0, 0)),
            pl.BlockSpec((256, 1), lambda i: (0, 0)),
            pl.BlockSpec((256, 256), lambda i: (0, 0)),
            pl.BlockSpec((256, 1), lambda i: (0, 0)),
        ],
        out_specs=pl.BlockSpec((2 * QB, 128), lambda i: (i, 0)),
        out_shape=jax.ShapeDtypeStruct((NBLK * 2 * QB, 128), jnp.float32),
    )(table_t, w1bd, b1bd, w2bd, b2bd)


def _sc_gather_body(idx_hbm, table_hbm, out_hbm,
                    idx_v, buf_a, buf_b, rows_t, sga, sgb):
    c = lax.axis_index("c")
    s = lax.axis_index("s")
    wid = s * NC + c
    base = wid * RPW
    pltpu.sync_copy(idx_hbm.at[wid], idx_v)

    lane = lax.iota(jnp.int32, 16)
    lane16 = lane + 16
    fpart0 = (lane >> 3) * 8192 + (lane & 7) * 128
    fpart1 = (lane16 >> 3) * 8192 + (lane16 & 7) * 128

    def fire(g, buf, sem):
        for i in range(GROUP):
            pltpu.async_copy(
                table_hbm.at[idx_v.at[g * GROUP + i]],
                buf.at[pl.ds(i * STEP, STEP)],
                sem)

    def drain(g, buf, sem):
        for i in range(GROUP):
            pltpu.make_async_copy(
                table_hbm.at[idx_v.at[g * GROUP + i]],
                buf.at[pl.ds(i * STEP, STEP)],
                sem).wait()

    def transpose_wb(g, buf):
        def tr(tc, carry):
            t0 = tc * 16
            tpart0 = ((t0 >> 7) << 10) + (t0 & 127)
            base0 = fpart0 + tpart0
            base1 = fpart1 + tpart0
            row0 = jnp.zeros((16,), jnp.int32) + t0
            for s in range(16):
                w = (lane + s) & 15
                rows = row0 + w
                v0 = plsc.load_gather(buf, [rows, lane])
                plsc.store_scatter(rows_t, [base0 + w], v0)
                v1 = plsc.load_gather(buf, [rows, lane16])
                plsc.store_scatter(rows_t, [base1 + w], v1)
            return carry

        lax.fori_loop(0, GR // 16, tr, 0)
        n0 = base + g * GR
        k = n0 // B
        w0 = (n0 % B) * 8
        for jt in range(4):
            pltpu.sync_copy(rows_t.at[pl.ds(jt * 8192, 8192)],
                            out_hbm.at[k, pl.ds(jt * 131072 + w0, 8192)])

    fire(0, buf_a, sga)

    def pair(u, carry):
        ga = 2 * u
        fire(ga + 1, buf_b, sgb)
        drain(ga, buf_a, sga)
        transpose_wb(ga, buf_a)
        fire(ga + 2, buf_a, sga)
        drain(ga + 1, buf_b, sgb)
        transpose_wb(ga + 1, buf_b)
        return carry

    lax.fori_loop(0, (NG - 1) // 2, pair, 0)
    drain(NG - 1, buf_a, sga)
    transpose_wb(NG - 1, buf_a)


def _sc_gather(idx3, table4):
    mesh = plsc.VectorSubcoreMesh(
        core_axis_name="c", subcore_axis_name="s", num_cores=NC, num_subcores=NS)
    return pl.kernel(
        _sc_gather_body,
        out_type=jax.ShapeDtypeStruct((K, D * B), jnp.float32),
        mesh=mesh,
        scratch_types=[
            pltpu.VMEM((S, STEP), jnp.int32),
            pltpu.VMEM((GR, D), jnp.float32),
            pltpu.VMEM((GR, D), jnp.float32),
            pltpu.VMEM((D * GR,), jnp.float32),
            pltpu.SemaphoreType.DMA,
            pltpu.SemaphoreType.DMA,
        ],
        compiler_params=pltpu.CompilerParams(
            use_tc_tiling_on_sc=False, needs_layout_passes=False,
            disable_bounds_checks=True),
    )(idx3, table4)


def kernel(attrs, attr_shift, unknown_index, table, W1, b1, W2, b2):
    attrs_t = attrs.T.astype(jnp.int32)
    unk = unknown_index.astype(jnp.int32)[:, None]
    r = jnp.where(attrs_t < 0, unk, attrs_t) + attr_shift.astype(jnp.int32).T
    rho = ((r >> 13) << 13) + ((r & 2047) << 2) + ((r >> 11) & 3)
    idx3 = rho.reshape(NW, S, STEP)

    eye8 = jnp.eye(8, dtype=jnp.float32)
    w1bd = jnp.kron(eye8, W1.T)
    w2bd = jnp.kron(eye8, W2.T)
    b1bd = jnp.tile(b1, 8).reshape(256, 1)
    b2bd = jnp.tile(b2, 8).reshape(256, 1)

    table2 = _tc_mlp_table(table.T, w1bd, b1bd, w2bd, b2bd)
    table4 = table2.reshape(T4, D)

    out2 = _sc_gather(idx3, table4)
    out5 = out2.reshape(K, 4, B // 128, 8, 128)
    return out5.transpose(2, 4, 0, 1, 3).reshape(B, K, D)
